# baseline (speedup 1.0000x reference)
import jax
import jax.numpy as jnp
from jax.experimental import pallas as pl
from jax.experimental.pallas import tpu as pltpu

_EMBEDDING_DIM = 128
_STEPS = 32
_N_FREQS = 16
_N_ATOMS = 256
_TIME = 1024
_BATCH = 2
_NB = 2 * _BATCH


def _pos_encode(x, n_freqs=_N_FREQS):
    outs = [x]
    for i in range(n_freqs):
        outs.append(jnp.sin((2.0 ** i) * x))
        outs.append(jnp.cos((2.0 ** i) * x))
    return jnp.concatenate(outs, axis=-1)


def _top32_body(a0_ref, a1_ref, b0_ref, b1_ref, vals_ref, aidx_ref, tidx_ref):
    refs = (a0_ref, a1_ref, b0_ref, b1_ref)
    am0 = tuple(jnp.max(r[...], axis=2) for r in refs)

    big = jnp.int32(1 << 30)
    it_a = (jax.lax.broadcasted_iota(jnp.int32, (2, 128), 0) * 128
            + jax.lax.broadcasted_iota(jnp.int32, (2, 128), 1))
    it_t = jax.lax.broadcasted_iota(jnp.int32, (1, 1, 1024), 2)
    lane = jax.lax.broadcasted_iota(jnp.int32, (1, _STEPS), 1)

    def step(i, carry):
        ams, vals, aidxs, tidxs = carry
        new_ams, new_vals, new_aidxs, new_tidxs = [], [], [], []
        for n in range(_NB):
            fm = refs[n]
            am = ams[n]
            m = jnp.max(am, axis=(0, 1), keepdims=True)
            a = jnp.min(jnp.where(am == m, it_a, big))
            a_hi = a // 128
            a_lo = a - a_hi * 128
            row = fm[pl.ds(a_hi, 1), pl.ds(a_lo, 1), :]
            m3 = m.reshape(1, 1, 1)
            t = jnp.min(jnp.where(row == m3, it_t, big), axis=(0, 1, 2),
                        keepdims=True)
            row2 = jnp.where(it_t == t, jnp.float32(0.0), row)
            fm[pl.ds(a_hi, 1), pl.ds(a_lo, 1), :] = row2
            m2 = jnp.max(row2, axis=(0, 1, 2), keepdims=True)
            new_ams.append(jnp.where(it_a == a, m2.reshape(1, 1), am))
            new_vals.append(jnp.where(lane == i, m.reshape(1, 1), vals[n]))
            new_aidxs.append(jnp.where(lane == i, a, aidxs[n]))
            new_tidxs.append(jnp.where(lane == i, t.reshape(1, 1), tidxs[n]))
        return (tuple(new_ams), tuple(new_vals), tuple(new_aidxs),
                tuple(new_tidxs))

    zf = [jnp.zeros((1, _STEPS), jnp.float32)] * _NB
    zi = [jnp.zeros((1, _STEPS), jnp.int32)] * _NB
    carry0 = (am0, tuple(zf), tuple(zi), tuple(zi))
    _, vals, aidx, tidx = jax.lax.fori_loop(0, _STEPS, step, carry0)
    for n in range(_NB):
        vals_ref[n] = vals[n]
        aidx_ref[n] = aidx[n]
        tidx_ref[n] = tidx[n]


def kernel(a, b, embeddings, ordering_w):
    a4 = a.reshape(_BATCH, 2, 128, _TIME)
    b4 = b.reshape(_BATCH, 2, 128, _TIME)
    vals, aidx, tidx = pl.pallas_call(
        _top32_body,
        out_shape=[
            jax.ShapeDtypeStruct((_NB, 1, _STEPS), jnp.float32),
            jax.ShapeDtypeStruct((_NB, 1, _STEPS), jnp.int32),
            jax.ShapeDtypeStruct((_NB, 1, _STEPS), jnp.int32),
        ],
    )(a4[0], a4[1], b4[0], b4[1])
    vals = vals.reshape(_NB, _STEPS)
    aidx = aidx.reshape(_NB, _STEPS)
    tidx = tidx.reshape(_NB, _STEPS)

    rng = jnp.linspace(0.0, 1.0, _TIME)
    scalar_pos = rng[tidx]
    pos_enc = _pos_encode(scalar_pos[..., None])
    v_enc = _pos_encode(vals[..., None])
    a_emb = embeddings[aidx]
    emb = jnp.concatenate([pos_enc, v_enc, a_emb], axis=-1)

    keys = emb @ ordering_w
    order = jnp.argsort(keys, axis=-1)
    emb = jnp.take_along_axis(emb, order[:, :, None], axis=1)
    ae, be = emb[:_BATCH], emb[_BATCH:]
    return jnp.mean((ae - be) ** 2)


# speedup vs baseline: 2.0177x; 2.0177x over previous
import jax
import jax.numpy as jnp
from jax.experimental import pallas as pl
from jax.experimental.pallas import tpu as pltpu

_EMBEDDING_DIM = 128
_STEPS = 32
_N_FREQS = 16
_N_ATOMS = 256
_TIME = 1024
_BATCH = 2
_NB = 2 * _BATCH
_NEG = float("-inf")


def _pos_encode(x, n_freqs=_N_FREQS):
    outs = [x]
    for i in range(n_freqs):
        outs.append(jnp.sin((2.0 ** i) * x))
        outs.append(jnp.cos((2.0 ** i) * x))
    return jnp.concatenate(outs, axis=-1)


def _exact_fallback(x_ref, vals_ref, aidx_ref, tidx_ref):
    big = jnp.int32(1 << 30)
    it_a = (jax.lax.broadcasted_iota(jnp.int32, (2, 128), 0) * 128
            + jax.lax.broadcasted_iota(jnp.int32, (2, 128), 1))
    it_t = jax.lax.broadcasted_iota(jnp.int32, (1, 1, 1024), 2)
    lane = jax.lax.broadcasted_iota(jnp.int32, (1, _STEPS), 1)
    am0 = tuple(jnp.max(x_ref[n], axis=2) for n in range(_NB))

    def step(i, carry):
        ams, vals, aidxs, tidxs = carry
        new = ([], [], [], [])
        for n in range(_NB):
            am = ams[n]
            m = jnp.max(am, axis=(0, 1), keepdims=True)
            a = jnp.min(jnp.where(am == m, it_a, big))
            a_hi = a // 128
            a_lo = a - a_hi * 128
            row = x_ref[n, pl.ds(a_hi, 1), pl.ds(a_lo, 1), :]
            t = jnp.min(jnp.where(row == m.reshape(1, 1, 1), it_t, big),
                        axis=(0, 1, 2), keepdims=True)
            row2 = jnp.where(it_t == t, jnp.float32(0.0), row)
            x_ref[n, pl.ds(a_hi, 1), pl.ds(a_lo, 1), :] = row2
            m2 = jnp.max(row2, axis=(0, 1, 2), keepdims=True)
            new[0].append(jnp.where(it_a == a, m2.reshape(1, 1), am))
            new[1].append(jnp.where(lane == i, m.reshape(1, 1), vals[n]))
            new[2].append(jnp.where(lane == i, a, aidxs[n]))
            new[3].append(jnp.where(lane == i, t.reshape(1, 1), tidxs[n]))
        return tuple(tuple(v) for v in new)

    zf = [jnp.zeros((1, _STEPS), jnp.float32)] * _NB
    zi = [jnp.zeros((1, _STEPS), jnp.int32)] * _NB
    _, vals, aidx, tidx = jax.lax.fori_loop(
        0, _STEPS, step, (am0, tuple(zf), tuple(zi), tuple(zi)))
    for n in range(_NB):
        vals_ref[n] = vals[n]
        aidx_ref[n] = aidx[n]
        tidx_ref[n] = tidx[n]


def _top32_body(x_ref, vals_ref, aidx_ref, tidx_ref):
    big = jnp.int32(1 << 30)
    it_t4 = jax.lax.broadcasted_iota(jnp.int32, (_NB, 2, 128, _TIME), 3)

    x = x_ref[...]
    vs, ts = [], []
    v_prev, t_prev = None, None
    for k in range(4):
        if k == 0:
            keep = None
            v_k = jnp.max(x, axis=3, keepdims=True)
        else:
            keep = (x < v_prev) | ((x == v_prev) & (it_t4 > t_prev))
            v_k = jnp.max(jnp.where(keep, x, _NEG), axis=3, keepdims=True)
        eq = (x == v_k) if keep is None else ((x == v_k) & keep)
        t_k = jnp.min(jnp.where(eq, it_t4, big), axis=3, keepdims=True)
        vs.append(v_k.reshape(_NB, 1, _N_ATOMS))
        ts.append(t_k.reshape(_NB, 1, _N_ATOMS))
        v_prev, t_prev = v_k, t_k

    it_a = jax.lax.broadcasted_iota(jnp.int32, (_NB, 1, _N_ATOMS), 2)
    lane = jax.lax.broadcasted_iota(jnp.int32, (_NB, 1, _STEPS), 2)

    def step(i, carry):
        (a1, a2, a3, a4, t1, t2, t3, t4, vals, aidx, tidx) = carry
        m = jnp.max(a1, axis=2, keepdims=True)
        a = jnp.min(jnp.where(a1 == m, it_a, big), axis=2, keepdims=True)
        oh = it_a == a
        t_e = jnp.min(jnp.where(oh, t1, big), axis=2, keepdims=True)
        vals = jnp.where(lane == i, m, vals)
        aidx = jnp.where(lane == i, a, aidx)
        tidx = jnp.where(lane == i, t_e, tidx)
        a1 = jnp.where(oh, a2, a1)
        t1 = jnp.where(oh, t2, t1)
        a2 = jnp.where(oh, a3, a2)
        t2 = jnp.where(oh, t3, t2)
        a3 = jnp.where(oh, a4, a3)
        t3 = jnp.where(oh, t4, t3)
        a4 = jnp.where(oh, jnp.float32(_NEG), a4)
        t4 = jnp.where(oh, big, t4)
        return (a1, a2, a3, a4, t1, t2, t3, t4, vals, aidx, tidx)

    carry0 = (vs[0], vs[1], vs[2], vs[3], ts[0], ts[1], ts[2], ts[3],
              jnp.zeros((_NB, 1, _STEPS), jnp.float32),
              jnp.zeros((_NB, 1, _STEPS), jnp.int32),
              jnp.zeros((_NB, 1, _STEPS), jnp.int32))
    out = jax.lax.fori_loop(0, _STEPS, step, carry0)
    a1_fin, vals, aidx, tidx = out[0], out[8], out[9], out[10]
    vals_ref[...] = vals
    aidx_ref[...] = aidx
    tidx_ref[...] = tidx

    bad = jnp.any(a1_fin == _NEG)

    @pl.when(bad)
    def _():
        _exact_fallback(x_ref, vals_ref, aidx_ref, tidx_ref)


def kernel(a, b, embeddings, ordering_w):
    x = jnp.concatenate([a, b], axis=0).reshape(_NB, 2, 128, _TIME)
    vals, aidx, tidx = pl.pallas_call(
        _top32_body,
        out_shape=[
            jax.ShapeDtypeStruct((_NB, 1, _STEPS), jnp.float32),
            jax.ShapeDtypeStruct((_NB, 1, _STEPS), jnp.int32),
            jax.ShapeDtypeStruct((_NB, 1, _STEPS), jnp.int32),
        ],
    )(x)
    vals = vals.reshape(_NB, _STEPS)
    aidx = aidx.reshape(_NB, _STEPS)
    tidx = tidx.reshape(_NB, _STEPS)

    rng = jnp.linspace(0.0, 1.0, _TIME)
    scalar_pos = rng[tidx]
    pos_enc = _pos_encode(scalar_pos[..., None])
    v_enc = _pos_encode(vals[..., None])
    a_emb = embeddings[aidx]
    emb = jnp.concatenate([pos_enc, v_enc, a_emb], axis=-1)

    keys = emb @ ordering_w
    order = jnp.argsort(keys, axis=-1)
    emb = jnp.take_along_axis(emb, order[:, :, None], axis=1)
    ae, be = emb[:_BATCH], emb[_BATCH:]
    return jnp.mean((ae - be) ** 2)


# speedup vs baseline: 2.6030x; 1.2901x over previous
import jax
import jax.numpy as jnp
from jax.experimental import pallas as pl
from jax.experimental.pallas import tpu as pltpu

_EMBEDDING_DIM = 128
_STEPS = 32
_N_FREQS = 16
_N_ATOMS = 256
_TIME = 1024
_BATCH = 2
_NB = 2 * _BATCH
_D_EMB = _EMBEDDING_DIM - 2 * (1 + 2 * _N_FREQS)
_NEG = float("-inf")


def _pos_encode(x, n_freqs=_N_FREQS):
    outs = [x]
    for i in range(n_freqs):
        outs.append(jnp.sin((2.0 ** i) * x))
        outs.append(jnp.cos((2.0 ** i) * x))
    return jnp.concatenate(outs, axis=-1)


def _exact_fallback(refs, vals_ref, aidx_ref, tidx_ref):
    big = jnp.int32(1 << 30)
    it_a = (jax.lax.broadcasted_iota(jnp.int32, (2, 128), 0) * 128
            + jax.lax.broadcasted_iota(jnp.int32, (2, 128), 1))
    it_t = jax.lax.broadcasted_iota(jnp.int32, (1, 1, 1024), 2)
    lane = jax.lax.broadcasted_iota(jnp.int32, (1, _STEPS), 1)
    am0 = tuple(jnp.max(r[n], axis=2) for r, n in refs)

    def step(i, carry):
        ams, vals, aidxs, tidxs = carry
        new = ([], [], [], [])
        for j, (r, n) in enumerate(refs):
            am = ams[j]
            m = jnp.max(am, axis=(0, 1), keepdims=True)
            a = jnp.min(jnp.where(am == m, it_a, big))
            a_hi = a // 128
            a_lo = a - a_hi * 128
            row = r[n, pl.ds(a_hi, 1), pl.ds(a_lo, 1), :]
            t = jnp.min(jnp.where(row == m.reshape(1, 1, 1), it_t, big),
                        axis=(0, 1, 2), keepdims=True)
            row2 = jnp.where(it_t == t, jnp.float32(0.0), row)
            r[n, pl.ds(a_hi, 1), pl.ds(a_lo, 1), :] = row2
            m2 = jnp.max(row2, axis=(0, 1, 2), keepdims=True)
            new[0].append(jnp.where(it_a == a, m2.reshape(1, 1), am))
            new[1].append(jnp.where(lane == i, m.reshape(1, 1), vals[j]))
            new[2].append(jnp.where(lane == i, a, aidxs[j]))
            new[3].append(jnp.where(lane == i, t.reshape(1, 1), tidxs[j]))
        return tuple(tuple(v) for v in new)

    zf = [jnp.zeros((1, _STEPS), jnp.float32)] * _NB
    zi = [jnp.zeros((1, _STEPS), jnp.int32)] * _NB
    _, vals, aidx, tidx = jax.lax.fori_loop(
        0, _STEPS, step, (am0, tuple(zf), tuple(zi), tuple(zi)))
    for j in range(_NB):
        vals_ref[j] = vals[j]
        aidx_ref[j] = aidx[j]
        tidx_ref[j] = tidx[j]


def _top32_body(a_ref, b_ref, vals_ref, aidx_ref, tidx_ref):
    big = jnp.int32(1 << 30)
    it_t4 = jax.lax.broadcasted_iota(jnp.int32, (_BATCH, 2, 128, _TIME), 3)

    def top4(x):
        vs, ts = [], []
        v_prev, t_prev = None, None
        for k in range(4):
            if k == 0:
                keep = None
                v_k = jnp.max(x, axis=3, keepdims=True)
            else:
                keep = (x < v_prev) | ((x == v_prev) & (it_t4 > t_prev))
                v_k = jnp.max(jnp.where(keep, x, _NEG), axis=3, keepdims=True)
            eq = (x == v_k) if keep is None else ((x == v_k) & keep)
            t_k = jnp.min(jnp.where(eq, it_t4, big), axis=3, keepdims=True)
            vs.append(v_k.reshape(_BATCH, 1, _N_ATOMS))
            ts.append(t_k.reshape(_BATCH, 1, _N_ATOMS))
            v_prev, t_prev = v_k, t_k
        return vs, ts

    avs, ats = top4(a_ref[...])
    bvs, bts = top4(b_ref[...])
    vs = [jnp.concatenate([av, bv], axis=0) for av, bv in zip(avs, bvs)]
    ts = [jnp.concatenate([at, bt], axis=0) for at, bt in zip(ats, bts)]

    it_a = jax.lax.broadcasted_iota(jnp.int32, (_NB, 1, _N_ATOMS), 2)
    lane = jax.lax.broadcasted_iota(jnp.int32, (_NB, 1, _STEPS), 2)

    def step(i, carry):
        (a1, a2, a3, a4, t1, t2, t3, t4, vals, aidx, tidx) = carry
        m = jnp.max(a1, axis=2, keepdims=True)
        a = jnp.min(jnp.where(a1 == m, it_a, big), axis=2, keepdims=True)
        oh = it_a == a
        t_e = jnp.min(jnp.where(oh, t1, big), axis=2, keepdims=True)
        vals = jnp.where(lane == i, m, vals)
        aidx = jnp.where(lane == i, a, aidx)
        tidx = jnp.where(lane == i, t_e, tidx)
        a1 = jnp.where(oh, a2, a1)
        t1 = jnp.where(oh, t2, t1)
        a2 = jnp.where(oh, a3, a2)
        t2 = jnp.where(oh, t3, t2)
        a3 = jnp.where(oh, a4, a3)
        t3 = jnp.where(oh, t4, t3)
        a4 = jnp.where(oh, jnp.float32(_NEG), a4)
        t4 = jnp.where(oh, big, t4)
        return (a1, a2, a3, a4, t1, t2, t3, t4, vals, aidx, tidx)

    carry0 = (vs[0], vs[1], vs[2], vs[3], ts[0], ts[1], ts[2], ts[3],
              jnp.zeros((_NB, 1, _STEPS), jnp.float32),
              jnp.zeros((_NB, 1, _STEPS), jnp.int32),
              jnp.zeros((_NB, 1, _STEPS), jnp.int32))
    out = jax.lax.fori_loop(0, _STEPS, step, carry0)
    a1_fin, vals, aidx, tidx = out[0], out[8], out[9], out[10]
    vals_ref[...] = vals
    aidx_ref[...] = aidx
    tidx_ref[...] = tidx

    bad = jnp.any(a1_fin == _NEG)

    @pl.when(bad)
    def _():
        refs = [(a_ref, 0), (a_ref, 1), (b_ref, 0), (b_ref, 1)]
        _exact_fallback(refs, vals_ref, aidx_ref, tidx_ref)


def _assemble_body(pos_ref, venc_ref, aidx_ref, emb_ref, w_ref, out_ref):
    it_atom = jax.lax.broadcasted_iota(jnp.int32, (_NB, _STEPS, _N_ATOMS), 2)
    aidx = aidx_ref[...].reshape(_NB, _STEPS, 1)
    oh = (it_atom == aidx).astype(jnp.float32)
    a_emb = jax.lax.dot_general(
        oh.reshape(_NB * _STEPS, _N_ATOMS), emb_ref[...],
        (((1,), (0,)), ((), ())),
        preferred_element_type=jnp.float32).reshape(_NB, _STEPS, _D_EMB)
    emb = jnp.concatenate([pos_ref[...], venc_ref[...], a_emb], axis=-1)

    keys = jnp.sum(emb * w_ref[...], axis=2, keepdims=True)
    keys_t = keys.reshape(_NB, 1, _STEPS)
    it_k = jax.lax.broadcasted_iota(jnp.int32, (_NB, _STEPS, 1), 1)
    it_kp = jax.lax.broadcasted_iota(jnp.int32, (_NB, 1, _STEPS), 2)
    less = keys_t < keys
    tie = (keys_t == keys) & (it_kp < it_k)
    ranks = jnp.sum((less | tie).astype(jnp.int32), axis=2, keepdims=True)
    perm = (ranks == it_kp).astype(jnp.float32)
    emb_sorted = jax.lax.dot_general(
        perm, emb, (((1,), (1,)), ((0,), (0,))),
        preferred_element_type=jnp.float32)
    diff = emb_sorted[:_BATCH] - emb_sorted[_BATCH:]
    total = jnp.sum(diff * diff, axis=(0, 1, 2), keepdims=True)
    out_ref[...] = total.reshape(1, 1) / (_BATCH * _STEPS * _EMBEDDING_DIM)


def kernel(a, b, embeddings, ordering_w):
    a4 = a.reshape(_BATCH, 2, 128, _TIME)
    b4 = b.reshape(_BATCH, 2, 128, _TIME)
    vals, aidx, tidx = pl.pallas_call(
        _top32_body,
        out_shape=[
            jax.ShapeDtypeStruct((_NB, 1, _STEPS), jnp.float32),
            jax.ShapeDtypeStruct((_NB, 1, _STEPS), jnp.int32),
            jax.ShapeDtypeStruct((_NB, 1, _STEPS), jnp.int32),
        ],
    )(a4, b4)

    rng = jnp.linspace(0.0, 1.0, _TIME)
    scalar_pos = rng[tidx.reshape(_NB, _STEPS)]
    pos_enc = _pos_encode(scalar_pos[..., None])
    v_enc = _pos_encode(vals.reshape(_NB, _STEPS)[..., None])

    out = pl.pallas_call(
        _assemble_body,
        out_shape=jax.ShapeDtypeStruct((1, 1), jnp.float32),
    )(pos_enc, v_enc, aidx, embeddings, ordering_w.reshape(1, 1, -1))
    return out.reshape(())
